# Initial kernel scaffold; baseline (speedup 1.0000x reference)
import jax
import jax.numpy as jnp
from jax.experimental import pallas as pl


def kernel(x, edge_index, W1_l, b1, W1_r, W2_l, b2, W2_r):
    raise NotImplementedError("write your pallas kernel here")


# speedup vs baseline: 3.4906x; 3.4906x over previous
import functools

import jax
import jax.numpy as jnp
from jax import lax
from jax.experimental import pallas as pl
from jax.experimental.pallas import tpu as pltpu
from jax.experimental.pallas import tpu_sc as plsc

N = 10000
D = 256
E = 160000
H = 128
NC, NS = 2, 16
NPAD = 10240
RPT = NPAD // NS
CHUNK = 64
CHUNKS = 158
EPAD = NS * CHUNKS * CHUNK


_mesh = plsc.VectorSubcoreMesh(core_axis_name="c", subcore_axis_name="s")


def _make_agg():
    scratch = [
        pltpu.VMEM((CHUNKS, CHUNK), jnp.int32),
        pltpu.VMEM((CHUNKS, CHUNK), jnp.int32),
        pltpu.VMEM((CHUNK, H), jnp.float32),
        pltpu.VMEM_SHARED((NPAD, H), jnp.float32),
    ]

    def body(table, srcs, dsts, zf, out, src_v, dst_v, rows_v, acc):
        cid = lax.axis_index("c")
        sid = lax.axis_index("s")
        base = sid * RPT
        pltpu.sync_copy(srcs.at[cid, sid], src_v)
        pltpu.sync_copy(dsts.at[sid], dst_v)
        pltpu.sync_copy(zf, acc.at[pl.ds(base, RPT)])
        plsc.subcore_barrier()

        def step(j, carry):
            pltpu.sync_copy(table.at[src_v.at[j]], rows_v)
            pltpu.sync_copy(rows_v, acc.at[dst_v.at[j]], add=True)
            return carry

        lax.fori_loop(0, CHUNKS, step, 0)
        plsc.subcore_barrier()
        pltpu.sync_copy(acc.at[pl.ds(base, RPT)], out.at[cid, pl.ds(base, RPT)])

    return pl.kernel(
        body,
        out_type=jax.ShapeDtypeStruct((NC, NPAD, H), jnp.float32),
        mesh=_mesh,
        scratch_types=scratch,
    )


def _make_deg():
    DCHUNKS = EPAD // (NC * NS * CHUNK)
    scratch = [
        pltpu.VMEM((DCHUNKS, CHUNK), jnp.int32),
        pltpu.VMEM((CHUNK, H), jnp.float32),
        pltpu.VMEM_SHARED((NPAD, H), jnp.float32),
    ]

    def body(dsts2, zd, ones, out, dst_v, ones_v, dacc):
        cid = lax.axis_index("c")
        sid = lax.axis_index("s")
        base = sid * RPT
        pltpu.sync_copy(dsts2.at[cid, sid], dst_v)
        pltpu.sync_copy(ones, ones_v)
        pltpu.sync_copy(zd, dacc.at[pl.ds(base, RPT)])
        plsc.subcore_barrier()

        def step(j, carry):
            pltpu.sync_copy(ones_v, dacc.at[dst_v.at[j]], add=True)
            return carry

        lax.fori_loop(0, DCHUNKS, step, 0)
        plsc.subcore_barrier()
        pltpu.sync_copy(dacc.at[pl.ds(base, RPT)], out.at[cid, pl.ds(base, RPT)])

    return pl.kernel(
        body,
        out_type=jax.ShapeDtypeStruct((NC, NPAD, H), jnp.float32),
        mesh=_mesh,
        scratch_types=scratch,
    )


_agg = _make_agg()
_deg = _make_deg()


def _make_combine(split_out):
    BN = 512
    grid = (NPAD // BN,)

    def body(agg_ref, deg_ref, x_ref, wl_ref, b_ref, wr_ref, o_ref):
        a = jnp.concatenate([agg_ref[0], agg_ref[1]], axis=1)
        xx = jnp.concatenate([x_ref[0], x_ref[1]], axis=1)
        d = deg_ref[0][:, 0:1] + deg_ref[1][:, 0:1]
        inv = 1.0 / jnp.maximum(d, 1.0)
        r = jnp.dot(a * inv, wl_ref[...], preferred_element_type=jnp.float32)
        r = r + jnp.dot(xx, wr_ref[...], preferred_element_type=jnp.float32)
        r = jnp.maximum(r + b_ref[...], 0.0)
        if split_out:
            o_ref[0] = r[:, :H]
            o_ref[1] = r[:, H:]
        else:
            o_ref[...] = r

    if split_out:
        out_spec = pl.BlockSpec((NC, BN, H), lambda i: (0, i, 0))
        out_shape = jax.ShapeDtypeStruct((NC, NPAD, H), jnp.float32)
    else:
        out_spec = pl.BlockSpec((BN, D), lambda i: (i, 0))
        out_shape = jax.ShapeDtypeStruct((NPAD, D), jnp.float32)

    return pl.pallas_call(
        body,
        grid=grid,
        in_specs=[
            pl.BlockSpec((NC, BN, H), lambda i: (0, i, 0)),
            pl.BlockSpec((NC, BN, H), lambda i: (0, i, 0)),
            pl.BlockSpec((NC, BN, H), lambda i: (0, i, 0)),
            pl.BlockSpec((D, D), lambda i: (0, 0)),
            pl.BlockSpec((1, D), lambda i: (0, 0)),
            pl.BlockSpec((D, D), lambda i: (0, 0)),
        ],
        out_specs=out_spec,
        out_shape=out_shape,
    )


_combine_split = _make_combine(True)
_combine_full = _make_combine(False)


@jax.jit
def kernel(x, edge_index, W1_l, b1, W1_r, W2_l, b2, W2_r):
    src = edge_index[0].astype(jnp.int32)
    dst = edge_index[1].astype(jnp.int32)
    pad = EPAD - E
    src_p = jnp.concatenate([src, jnp.zeros((pad,), jnp.int32)])
    dst_p = jnp.concatenate([dst, jnp.full((pad,), N, jnp.int32)])
    srcs = jnp.stack([src_p, src_p + NPAD]).reshape(NC, NS, CHUNKS, CHUNK)
    dsts = dst_p.reshape(NS, CHUNKS, CHUNK)
    dsts2 = dst_p.reshape(NC, NS, CHUNKS // NC, CHUNK)

    x_pad = jnp.pad(x, ((0, NPAD - N), (0, 0)))
    table1 = x_pad.reshape(NPAD, NC, H).transpose(1, 0, 2)

    zf = jnp.zeros((RPT, H), jnp.float32)
    ones = jnp.ones((CHUNK, H), jnp.float32)

    deg = _deg(dsts2, zf, ones)
    agg1 = _agg(table1.reshape(NC * NPAD, H), srcs, dsts, zf)
    h = _combine_split(agg1, deg, table1, W1_l.T, b1[None, :], W1_r.T)
    agg2 = _agg(h.reshape(NC * NPAD, H), srcs, dsts, zf)
    out = _combine_full(agg2, deg, h, W2_l.T, b2[None, :], W2_r.T)
    return out[:N]
